# baseline (speedup 1.0000x reference)
import jax
import jax.numpy as jnp
from jax.experimental import pallas as pl
from jax.scipy.special import logsumexp

HIDDEN_DIM = 1024
NUM_HEADS = 16
HEAD_DIM = 64
CHUNK_LEN = 128
N_HASHES = 2
N_BUCKETS = 64


def _hash_vectors(vecs, rng, num_buckets, num_hashes):
    rot_size = num_buckets
    rotations_shape = (vecs.shape[-1], num_hashes, rot_size // 2)
    random_rotations = jax.random.normal(rng, rotations_shape).astype(jnp.float32)
    rotated_vecs = jnp.einsum('tf,fhi->hti', vecs, random_rotations)
    rotated_vecs = jnp.concatenate([rotated_vecs, -rotated_vecs], axis=-1)
    buckets = jnp.argmax(rotated_vecs, axis=-1)
    offsets = jnp.reshape(jnp.arange(num_hashes) * num_buckets, (-1, 1))
    buckets = jnp.reshape(buckets + offsets, (-1,))
    return buckets


def _look_one_back(x):
    xlb = jnp.concatenate([x[-1:, ...], x[:-1, ...]], axis=0)
    return jnp.concatenate([x, xlb], axis=1)


def _lsh_attention_single_head(query, value, n_buckets, n_hashes):
    qdim = query.shape[-1]
    chunk_size = n_hashes * n_buckets
    seqlen = query.shape[0]
    rng = jax.random.PRNGKey(0)
    buckets = _hash_vectors(query, rng, num_buckets=n_buckets, num_hashes=n_hashes)
    ticker = jnp.arange(n_hashes * seqlen)
    buckets_and_t = seqlen * buckets + ticker % seqlen
    _, sticker = jax.lax.sort_key_val(buckets_and_t, ticker, dimension=-1)
    _, undo_sort = jax.lax.sort_key_val(sticker, ticker, dimension=-1)
    st = sticker % seqlen
    sqk = jnp.take(query, st, axis=0)
    sv = jnp.take(value, st, axis=0)
    bqk = jnp.reshape(sqk, (chunk_size, -1, qdim))
    bv = jnp.reshape(sv, (chunk_size, -1, qdim))
    bq = bqk
    bk = _look_one_back(bqk)
    bv = _look_one_back(bv)
    dots = jnp.einsum('hie,hje->hij', bq, bk) * qdim ** 0.5
    dots_logsumexp = logsumexp(dots, axis=-1, keepdims=True)
    slogits = jnp.reshape(dots_logsumexp, [-1])
    dots = jnp.exp(dots - dots_logsumexp)
    x = jnp.matmul(dots, bv)
    x = jnp.reshape(x, [-1, qdim])
    o = jnp.take(x, undo_sort, axis=0)
    _, logits = jax.lax.sort_key_val(sticker, slogits, dimension=-1)
    logits = jnp.reshape(logits, [n_hashes, seqlen, 1])
    probs = jnp.exp(logits - logsumexp(logits, axis=0, keepdims=True))
    o = jnp.reshape(o, [n_hashes, seqlen, qdim])
    out = jnp.sum(o * probs, axis=0)
    return jnp.reshape(out, [seqlen, qdim])


def kernel(inputs_q, inputs_k, inputs_v, q_kernel, q_bias, k_kernel, k_bias,
           v_kernel, v_bias, out_kernel, out_bias, step):
    del step, inputs_k, k_kernel, k_bias
    qlength = inputs_q.shape[1]
    extra_len = CHUNK_LEN - qlength % CHUNK_LEN
    pad_width = ((0, 0), (0, extra_len), (0, 0))
    q_in = jnp.pad(inputs_q, pad_width)
    v_in = jnp.pad(inputs_v, pad_width)
    query = jnp.einsum('bld,dhk->blhk', q_in, q_kernel) + q_bias
    value = jnp.einsum('bld,dhk->blhk', v_in, v_kernel) + v_bias

    def single_batch(q, v):
        attn = jax.vmap(_lsh_attention_single_head, in_axes=(1, 1, None, None))
        return attn(q, v, N_BUCKETS, N_HASHES)

    out = jax.vmap(single_batch, in_axes=(0, 0))(query, value)
    out = jnp.transpose(out, (0, 2, 1, 3))
    out = out[:, :qlength, :, :]
    return jnp.einsum('blhk,hkd->bld', out, out_kernel) + out_bias


# speedup vs baseline: 5.1749x; 5.1749x over previous
import functools

import jax
import jax.numpy as jnp
from jax import lax
from jax.experimental import pallas as pl
from jax.experimental.pallas import tpu as pltpu
from jax.experimental.pallas import tpu_sc as plsc

HIDDEN_DIM = 1024
NUM_HEADS = 16
HEAD_DIM = 64
CHUNK_LEN = 128
N_HASHES = 2
N_BUCKETS = 64

H = NUM_HEADS
K = HEAD_DIM
D = HIDDEN_DIM
T = 4224
S = N_HASHES * T
NCH = S // CHUNK_LEN
TCH = T // CHUNK_LEN
RW = 2 * K


def _proj_body(xq_ref, xv_ref, wq_ref, bq_ref, wv_ref, bv_ref, qv_out):
    x = xq_ref[0]
    q = jnp.dot(x, wq_ref[...], preferred_element_type=jnp.float32) + bq_ref[...]
    xv = xv_ref[0]
    v = jnp.dot(xv, wv_ref[...], preferred_element_type=jnp.float32) + bv_ref[...]
    for h in range(H):
        qv_out[0, h, :, 0:K] = q[:, h * K:(h + 1) * K]
        qv_out[0, h, :, K:RW] = v[:, h * K:(h + 1) * K]


def _stage_a(xq, xv, wq, bq, wv, bv, interpret=False):
    B = xq.shape[0]
    TB = 528
    grid = (B, T // TB)
    return pl.pallas_call(
        _proj_body,
        grid=grid,
        in_specs=[
            pl.BlockSpec((1, TB, D), lambda b, s: (b, s, 0)),
            pl.BlockSpec((1, TB, D), lambda b, s: (b, s, 0)),
            pl.BlockSpec((D, D), lambda b, s: (0, 0)),
            pl.BlockSpec((1, D), lambda b, s: (0, 0)),
            pl.BlockSpec((D, D), lambda b, s: (0, 0)),
            pl.BlockSpec((1, D), lambda b, s: (0, 0)),
        ],
        out_specs=pl.BlockSpec((1, H, TB, RW), lambda b, s: (b, 0, s, 0)),
        out_shape=jax.ShapeDtypeStruct((B, H, T, RW), jnp.float32),
        interpret=interpret,
    )(xq, xv, wq, bq, wv, bv)


def _sort_body(qv_ref, r_ref, pos_ref, oh_ref, bo_ref):
    f32 = jnp.float32
    q = qv_ref[0, 0][:, 0:K]
    rot = jnp.dot(q, r_ref[...], preferred_element_type=f32)

    r64 = lax.broadcasted_iota(jnp.int32, (64, 64), 0)
    c64 = lax.broadcasted_iota(jnp.int32, (64, 64), 1)
    u_incl = (r64 <= c64).astype(f32)

    zeros_t = jnp.zeros((T, 64), f32)
    for j in range(N_HASHES):
        rj = rot[:, 32 * j:32 * (j + 1)]
        vals = jnp.concatenate([rj, -rj], axis=1)
        m = jnp.max(vals, axis=1, keepdims=True)
        oh = (vals == m).astype(f32)
        cum = jnp.dot(oh, u_incl, preferred_element_type=f32)
        first = oh * (cum == 1.0).astype(f32)
        oh_ref[pl.ds(j * T, T), pl.ds(j * 64, 64)] = first
        oh_ref[pl.ds(j * T, T), pl.ds((1 - j) * 64, 64)] = zeros_t

    r128 = lax.broadcasted_iota(jnp.int32, (128, 128), 0)
    c128 = lax.broadcasted_iota(jnp.int32, (128, 128), 1)
    l_strict = (c128 < r128).astype(f32)
    u_strict = (r128 < c128).astype(f32)

    def body1(j, running):
        ohj = oh_ref[pl.ds(j * 128, 128), :]
        bo_ref[pl.ds(j, 1), :] = running
        return running + jnp.sum(ohj, axis=0, keepdims=True)

    total = lax.fori_loop(0, NCH, body1, jnp.zeros((1, 128), f32))
    starts = jnp.dot(total, u_strict, preferred_element_type=f32)

    def body2(j, _):
        ohj = oh_ref[pl.ds(j * 128, 128), :]
        pe = jnp.dot(l_strict, ohj, preferred_element_type=f32)
        pe = pe + bo_ref[pl.ds(j, 1), :] + starts
        posj = jnp.sum(ohj * pe, axis=1, keepdims=True)
        pos_ref[0, 0, pl.ds(j * 128, 128), :] = posj.astype(jnp.int32)
        return 0

    lax.fori_loop(0, NCH, body2, 0)


def _stage_b(qv_bhtk, rots, interpret=False):
    B = qv_bhtk.shape[0]
    grid = (B * H,)
    return pl.pallas_call(
        _sort_body,
        grid=grid,
        in_specs=[
            pl.BlockSpec((1, 1, T, RW), lambda w: (w // H, w % H, 0, 0)),
            pl.BlockSpec((K, 64), lambda w: (0, 0)),
        ],
        out_specs=pl.BlockSpec((1, 1, S, 1), lambda w: (w // H, w % H, 0, 0)),
        out_shape=jax.ShapeDtypeStruct((B, H, S, 1), jnp.int32),
        scratch_shapes=[
            pltpu.VMEM((S, 128), jnp.float32),
            pltpu.VMEM((NCH, 128), jnp.float32),
        ],
        interpret=interpret,
    )(qv_bhtk, rots)


def _stage_c_sc(qv_flat, pos_flat):
    BH = qv_flat.shape[0] // T
    mesh = plsc.VectorSubcoreMesh(core_axis_name="c", subcore_axis_name="s")

    @functools.partial(
        pl.kernel,
        mesh=mesh,
        out_type=jax.ShapeDtypeStruct((BH * S, RW), jnp.float32),
        scratch_types=[
            pltpu.VMEM((128,), jnp.int32),
            pltpu.VMEM((1, 128), jnp.int32),
            pltpu.VMEM((128, RW), jnp.float32),
            pltpu.SemaphoreType.DMA,
        ],
    )
    def sc_fwd(qv_hbm, pos_hbm, sqv_hbm, idx_raw, idx_adj, rows, sem):
        wid = lax.axis_index("s") * 2 + lax.axis_index("c")
        base_s = wid * S
        base_t = wid * T

        def chunk(c, _):
            pltpu.sync_copy(pos_hbm.at[pl.ds(base_s + c * 128, 128)], idx_raw)

            def add16(k2, _):
                idx_adj[0, pl.ds(k2 * 16, 16)] = (
                    idx_raw[pl.ds(k2 * 16, 16)] + base_s)
                return 0

            lax.fori_loop(0, 8, add16, 0)
            src = base_t + lax.rem(c, TCH) * 128
            pltpu.sync_copy(qv_hbm.at[pl.ds(src, 128)], rows)
            cp = pltpu.make_async_copy(rows, sqv_hbm.at[idx_adj.at[0]], sem)
            cp.start()
            cp.wait()
            return 0

        lax.fori_loop(0, NCH, chunk, 0)

    return sc_fwd(qv_flat, pos_flat)


def _attn_body(sqv_ref, xl_ref):
    def body(c, _):
        p = lax.rem(c + NCH - 1, NCH)
        cur = sqv_ref[0, pl.ds(c * 128, 128), :]
        prv = sqv_ref[0, pl.ds(p * 128, 128), :]
        bq = cur[:, 0:K]
        bk = jnp.concatenate([cur[:, 0:K], prv[:, 0:K]], axis=0)
        bv = jnp.concatenate([cur[:, K:RW], prv[:, K:RW]], axis=0)
        dots = lax.dot_general(bq, bk, (((1,), (1,)), ((), ())),
                               preferred_element_type=jnp.float32) * 8.0
        m = jnp.max(dots, axis=1, keepdims=True)
        lse = m + jnp.log(jnp.sum(jnp.exp(dots - m), axis=1, keepdims=True))
        pr = jnp.exp(dots - lse)
        xo = jnp.dot(pr, bv, preferred_element_type=jnp.float32)
        xl_ref[0, pl.ds(c * 128, 128), 0:K] = xo
        xl_ref[0, pl.ds(c * 128, 128), K:RW] = jnp.broadcast_to(lse, (128, K))
        return 0

    lax.fori_loop(0, NCH, body, 0)


def _stage_d(sqv, interpret=False):
    BH = sqv.shape[0]
    return pl.pallas_call(
        _attn_body,
        grid=(BH,),
        in_specs=[pl.BlockSpec((1, S, RW), lambda w: (w, 0, 0))],
        out_specs=pl.BlockSpec((1, S, RW), lambda w: (w, 0, 0)),
        out_shape=jax.ShapeDtypeStruct((BH, S, RW), jnp.float32),
        interpret=interpret,
    )(sqv)


def _stage_e_sc(xl_flat, pos_flat):
    BH = xl_flat.shape[0] // S
    mesh = plsc.VectorSubcoreMesh(core_axis_name="c", subcore_axis_name="s")

    @functools.partial(
        pl.kernel,
        mesh=mesh,
        out_type=jax.ShapeDtypeStruct((BH * S, RW), jnp.float32),
        scratch_types=[
            pltpu.VMEM((128,), jnp.int32),
            pltpu.VMEM((1, 128), jnp.int32),
            pltpu.VMEM((128, RW), jnp.float32),
            pltpu.SemaphoreType.DMA,
        ],
    )
    def sc_bwd(xl_hbm, pos_hbm, ol_hbm, idx_raw, idx_adj, rows, sem):
        wid = lax.axis_index("s") * 2 + lax.axis_index("c")
        base_s = wid * S

        def chunk(c, _):
            pltpu.sync_copy(pos_hbm.at[pl.ds(base_s + c * 128, 128)], idx_raw)

            def add16(k2, _):
                idx_adj[0, pl.ds(k2 * 16, 16)] = (
                    idx_raw[pl.ds(k2 * 16, 16)] + base_s)
                return 0

            lax.fori_loop(0, 8, add16, 0)
            g = pltpu.make_async_copy(xl_hbm.at[idx_adj.at[0]], rows, sem)
            g.start()
            g.wait()
            pltpu.sync_copy(rows, ol_hbm.at[pl.ds(base_s + c * 128, 128)])
            return 0

        lax.fori_loop(0, NCH, chunk, 0)

    return sc_bwd(xl_flat, pos_flat)


def _out_body(ol_ref, w_ref, b_ref, out_ref):
    h = pl.program_id(2)
    o0 = ol_ref[0, 0, :, 0:K]
    o1 = ol_ref[0, 1, :, 0:K]
    l0 = ol_ref[0, 0, :, K:K + 1]
    l1 = ol_ref[0, 1, :, K:K + 1]
    m = jnp.maximum(l0, l1)
    e0 = jnp.exp(l0 - m)
    e1 = jnp.exp(l1 - m)
    comb = (o0 * e0 + o1 * e1) / (e0 + e1)
    part = jnp.dot(comb, w_ref[0], preferred_element_type=jnp.float32)

    @pl.when(h == 0)
    def _():
        out_ref[0] = part + b_ref[...]

    @pl.when(h != 0)
    def _():
        out_ref[0] = out_ref[0] + part


def _stage_f(ol4, w_out, b_out, qlen, interpret=False):
    BH = ol4.shape[0]
    B = BH // H
    TBF = 512
    grid = (B, qlen // TBF, H)
    return pl.pallas_call(
        _out_body,
        grid=grid,
        in_specs=[
            pl.BlockSpec((1, 2, TBF, RW), lambda b, s, h: (b * H + h, 0, s, 0)),
            pl.BlockSpec((1, K, D), lambda b, s, h: (h, 0, 0)),
            pl.BlockSpec((1, D), lambda b, s, h: (0, 0)),
        ],
        out_specs=pl.BlockSpec((1, TBF, D), lambda b, s, h: (b, s, 0)),
        out_shape=jax.ShapeDtypeStruct((B, qlen, D), jnp.float32),
        interpret=interpret,
    )(ol4, w_out, b_out)


def kernel(inputs_q, inputs_k, inputs_v, q_kernel, q_bias, k_kernel, k_bias,
           v_kernel, v_bias, out_kernel, out_bias, step):
    del step, inputs_k, k_kernel, k_bias
    B, qlen, _ = inputs_q.shape
    extra = CHUNK_LEN - qlen % CHUNK_LEN
    pad = ((0, 0), (0, extra), (0, 0))
    xq = jnp.pad(inputs_q, pad)
    xv = jnp.pad(inputs_v, pad)

    rots = jax.random.normal(
        jax.random.PRNGKey(0),
        (K, N_HASHES, N_BUCKETS // 2)).astype(jnp.float32).reshape(K, 64)

    wq = q_kernel.reshape(D, H * K)
    wv = v_kernel.reshape(D, H * K)
    bq = q_bias.reshape(1, H * K)
    bv = v_bias.reshape(1, H * K)

    qv_bhtk = _stage_a(xq, xv, wq, bq, wv, bv)
    pos = _stage_b(qv_bhtk, rots)

    qv_flat = qv_bhtk.reshape(B * H * T, RW)
    pos_flat = pos.reshape(B * H * S)

    sqv_flat = _stage_c_sc(qv_flat, pos_flat)

    xl_bh = _stage_d(sqv_flat.reshape(B * H, S, RW))

    ol_flat = _stage_e_sc(xl_bh.reshape(B * H * S, RW), pos_flat)

    ol4 = ol_flat.reshape(B * H, N_HASHES, T, RW)
    w_out = out_kernel.reshape(H, K, D)
    b_out = out_bias.reshape(1, D)

    return _stage_f(ol4, w_out, b_out, qlen)


# speedup vs baseline: 7.7736x; 1.5022x over previous
import functools

import jax
import jax.numpy as jnp
from jax import lax
from jax.experimental import pallas as pl
from jax.experimental.pallas import tpu as pltpu
from jax.experimental.pallas import tpu_sc as plsc

HIDDEN_DIM = 1024
NUM_HEADS = 16
HEAD_DIM = 64
CHUNK_LEN = 128
N_HASHES = 2
N_BUCKETS = 64

H = NUM_HEADS
K = HEAD_DIM
D = HIDDEN_DIM
T = 4224
S = N_HASHES * T
NCH = S // CHUNK_LEN
TCH = T // CHUNK_LEN
RW = 2 * K


def _proj_body(xq_ref, xv_ref, wq_ref, bq_ref, wv_ref, bv_ref, qv_out):
    x = xq_ref[0]
    q = jnp.dot(x, wq_ref[...], preferred_element_type=jnp.float32) + bq_ref[...]
    xv = xv_ref[0]
    v = jnp.dot(xv, wv_ref[...], preferred_element_type=jnp.float32) + bv_ref[...]
    for h in range(H):
        qv_out[0, h, :, 0:K] = q[:, h * K:(h + 1) * K]
        qv_out[0, h, :, K:RW] = v[:, h * K:(h + 1) * K]


def _stage_a(xq, xv, wq, bq, wv, bv, interpret=False):
    B = xq.shape[0]
    TB = 528
    grid = (B, T // TB)
    return pl.pallas_call(
        _proj_body,
        grid=grid,
        in_specs=[
            pl.BlockSpec((1, TB, D), lambda b, s: (b, s, 0)),
            pl.BlockSpec((1, TB, D), lambda b, s: (b, s, 0)),
            pl.BlockSpec((D, D), lambda b, s: (0, 0)),
            pl.BlockSpec((1, D), lambda b, s: (0, 0)),
            pl.BlockSpec((D, D), lambda b, s: (0, 0)),
            pl.BlockSpec((1, D), lambda b, s: (0, 0)),
        ],
        out_specs=pl.BlockSpec((1, H, TB, RW), lambda b, s: (b, 0, s, 0)),
        out_shape=jax.ShapeDtypeStruct((B, H, T, RW), jnp.float32),
        interpret=interpret,
    )(xq, xv, wq, bq, wv, bv)


def _sort_body(qv_ref, r_ref, pos_ref):
    f32 = jnp.float32
    q = qv_ref[0, 0][:, 0:K]
    rot = jnp.dot(q, r_ref[...], preferred_element_type=f32)

    r64 = lax.broadcasted_iota(jnp.int32, (64, 64), 0)
    c64 = lax.broadcasted_iota(jnp.int32, (64, 64), 1)
    u_incl = (r64 <= c64).astype(f32)
    u_strict = (r64 < c64).astype(f32)
    r128 = lax.broadcasted_iota(jnp.int32, (128, 128), 0)
    c128 = lax.broadcasted_iota(jnp.int32, (128, 128), 1)
    l_strict = (c128 < r128).astype(f32)

    for j in range(N_HASHES):
        rj = rot[:, 32 * j:32 * (j + 1)]
        vals = jnp.concatenate([rj, -rj], axis=1)
        m = jnp.max(vals, axis=1, keepdims=True)
        oh = (vals == m).astype(f32)
        cum = jnp.dot(oh, u_incl, preferred_element_type=f32)
        first = oh * (cum == 1.0).astype(f32)

        running = jnp.zeros((1, 64), f32)
        offs = []
        for c in range(TCH):
            ohc = first[c * 128:(c + 1) * 128, :]
            offs.append(running)
            running = running + jnp.sum(ohc, axis=0, keepdims=True)
        starts = jnp.dot(running, u_strict, preferred_element_type=f32)
        starts = starts + jnp.float32(j * T)
        for c in range(TCH):
            ohc = first[c * 128:(c + 1) * 128, :]
            pe = jnp.dot(l_strict, ohc, preferred_element_type=f32)
            pe = pe + offs[c] + starts
            posc = jnp.sum(ohc * pe, axis=1, keepdims=True)
            pos_ref[0, 0, pl.ds(j * T + c * 128, 128), :] = posc.astype(
                jnp.int32)


def _stage_b(qv_bhtk, rots, interpret=False):
    B = qv_bhtk.shape[0]
    grid = (B * H,)
    return pl.pallas_call(
        _sort_body,
        grid=grid,
        in_specs=[
            pl.BlockSpec((1, 1, T, RW), lambda w: (w // H, w % H, 0, 0)),
            pl.BlockSpec((K, 64), lambda w: (0, 0)),
        ],
        out_specs=pl.BlockSpec((1, 1, S, 1), lambda w: (w // H, w % H, 0, 0)),
        out_shape=jax.ShapeDtypeStruct((B, H, S, 1), jnp.int32),
        interpret=interpret,
    )(qv_bhtk, rots)


def _stage_c_sc(qv_flat, pos_flat):
    BH = qv_flat.shape[0] // T
    mesh = plsc.VectorSubcoreMesh(core_axis_name="c", subcore_axis_name="s")

    @functools.partial(
        pl.kernel,
        mesh=mesh,
        out_type=jax.ShapeDtypeStruct((BH * S, RW), jnp.float32),
        scratch_types=[
            pltpu.VMEM((128,), jnp.int32),
            pltpu.VMEM((1, 128), jnp.int32),
            pltpu.VMEM((128, RW), jnp.float32),
            pltpu.SemaphoreType.DMA,
        ],
    )
    def sc_fwd(qv_hbm, pos_hbm, sqv_hbm, idx_raw, idx_adj, rows, sem):
        wid = lax.axis_index("s") * 2 + lax.axis_index("c")
        base_s = wid * S
        base_t = wid * T

        def chunk(c, _):
            pltpu.sync_copy(pos_hbm.at[pl.ds(base_s + c * 128, 128)], idx_raw)

            def add16(k2, _):
                idx_adj[0, pl.ds(k2 * 16, 16)] = (
                    idx_raw[pl.ds(k2 * 16, 16)] + base_s)
                return 0

            lax.fori_loop(0, 8, add16, 0)
            src = base_t + lax.rem(c, TCH) * 128
            pltpu.sync_copy(qv_hbm.at[pl.ds(src, 128)], rows)
            cp = pltpu.make_async_copy(rows, sqv_hbm.at[idx_adj.at[0]], sem)
            cp.start()
            cp.wait()
            return 0

        lax.fori_loop(0, NCH, chunk, 0)

    return sc_fwd(qv_flat, pos_flat)


def _attn_body(sqv_ref, xl_ref):
    for c in range(NCH):
        p = (c + NCH - 1) % NCH
        cur = sqv_ref[0, pl.ds(c * 128, 128), :]
        prv = sqv_ref[0, pl.ds(p * 128, 128), :]
        bq = cur[:, 0:K]
        d1 = lax.dot_general(bq, cur[:, 0:K], (((1,), (1,)), ((), ())),
                             preferred_element_type=jnp.float32) * 8.0
        d2 = lax.dot_general(bq, prv[:, 0:K], (((1,), (1,)), ((), ())),
                             preferred_element_type=jnp.float32) * 8.0
        m = jnp.maximum(jnp.max(d1, axis=1, keepdims=True),
                        jnp.max(d2, axis=1, keepdims=True))
        e1 = jnp.exp(d1 - m)
        e2 = jnp.exp(d2 - m)
        ssum = (jnp.sum(e1, axis=1, keepdims=True)
                + jnp.sum(e2, axis=1, keepdims=True))
        lse = m + jnp.log(ssum)
        inv = 1.0 / ssum
        xo = (jnp.dot(e1, cur[:, K:RW], preferred_element_type=jnp.float32)
              + jnp.dot(e2, prv[:, K:RW], preferred_element_type=jnp.float32)
              ) * inv
        xl_ref[0, pl.ds(c * 128, 128), 0:K] = xo
        xl_ref[0, pl.ds(c * 128, 128), K:RW] = jnp.broadcast_to(lse, (128, K))


def _stage_d(sqv, interpret=False):
    BH = sqv.shape[0]
    return pl.pallas_call(
        _attn_body,
        grid=(BH,),
        in_specs=[pl.BlockSpec((1, S, RW), lambda w: (w, 0, 0))],
        out_specs=pl.BlockSpec((1, S, RW), lambda w: (w, 0, 0)),
        out_shape=jax.ShapeDtypeStruct((BH, S, RW), jnp.float32),
        interpret=interpret,
    )(sqv)


def _stage_e_sc(xl_flat, pos_flat):
    BH = xl_flat.shape[0] // S
    mesh = plsc.VectorSubcoreMesh(core_axis_name="c", subcore_axis_name="s")

    @functools.partial(
        pl.kernel,
        mesh=mesh,
        out_type=jax.ShapeDtypeStruct((BH * S, RW), jnp.float32),
        scratch_types=[
            pltpu.VMEM((128,), jnp.int32),
            pltpu.VMEM((1, 128), jnp.int32),
            pltpu.VMEM((128, RW), jnp.float32),
            pltpu.SemaphoreType.DMA,
        ],
    )
    def sc_bwd(xl_hbm, pos_hbm, ol_hbm, idx_raw, idx_adj, rows, sem):
        wid = lax.axis_index("s") * 2 + lax.axis_index("c")
        base_s = wid * S

        def chunk(c, _):
            pltpu.sync_copy(pos_hbm.at[pl.ds(base_s + c * 128, 128)], idx_raw)

            def add16(k2, _):
                idx_adj[0, pl.ds(k2 * 16, 16)] = (
                    idx_raw[pl.ds(k2 * 16, 16)] + base_s)
                return 0

            lax.fori_loop(0, 8, add16, 0)
            g = pltpu.make_async_copy(xl_hbm.at[idx_adj.at[0]], rows, sem)
            g.start()
            g.wait()
            pltpu.sync_copy(rows, ol_hbm.at[pl.ds(base_s + c * 128, 128)])
            return 0

        lax.fori_loop(0, NCH, chunk, 0)

    return sc_bwd(xl_flat, pos_flat)


def _out_body(ol_ref, w_ref, b_ref, out_ref):
    h = pl.program_id(2)
    o0 = ol_ref[0, 0, :, 0:K]
    o1 = ol_ref[0, 1, :, 0:K]
    l0 = ol_ref[0, 0, :, K:K + 1]
    l1 = ol_ref[0, 1, :, K:K + 1]
    m = jnp.maximum(l0, l1)
    e0 = jnp.exp(l0 - m)
    e1 = jnp.exp(l1 - m)
    comb = (o0 * e0 + o1 * e1) / (e0 + e1)
    part = jnp.dot(comb, w_ref[0], preferred_element_type=jnp.float32)

    @pl.when(h == 0)
    def _():
        out_ref[0] = part + b_ref[...]

    @pl.when(h != 0)
    def _():
        out_ref[0] = out_ref[0] + part


def _stage_f(ol4, w_out, b_out, qlen, interpret=False):
    BH = ol4.shape[0]
    B = BH // H
    TBF = 512
    grid = (B, qlen // TBF, H)
    return pl.pallas_call(
        _out_body,
        grid=grid,
        in_specs=[
            pl.BlockSpec((1, 2, TBF, RW), lambda b, s, h: (b * H + h, 0, s, 0)),
            pl.BlockSpec((1, K, D), lambda b, s, h: (h, 0, 0)),
            pl.BlockSpec((1, D), lambda b, s, h: (0, 0)),
        ],
        out_specs=pl.BlockSpec((1, TBF, D), lambda b, s, h: (b, s, 0)),
        out_shape=jax.ShapeDtypeStruct((B, qlen, D), jnp.float32),
        interpret=interpret,
    )(ol4, w_out, b_out)


def kernel(inputs_q, inputs_k, inputs_v, q_kernel, q_bias, k_kernel, k_bias,
           v_kernel, v_bias, out_kernel, out_bias, step):
    del step, inputs_k, k_kernel, k_bias
    B, qlen, _ = inputs_q.shape
    extra = CHUNK_LEN - qlen % CHUNK_LEN
    pad = ((0, 0), (0, extra), (0, 0))
    xq = jnp.pad(inputs_q, pad)
    xv = jnp.pad(inputs_v, pad)

    rots = jax.random.normal(
        jax.random.PRNGKey(0),
        (K, N_HASHES, N_BUCKETS // 2)).astype(jnp.float32).reshape(K, 64)

    wq = q_kernel.reshape(D, H * K)
    wv = v_kernel.reshape(D, H * K)
    bq = q_bias.reshape(1, H * K)
    bv = v_bias.reshape(1, H * K)

    qv_bhtk = _stage_a(xq, xv, wq, bq, wv, bv)
    pos = _stage_b(qv_bhtk, rots)

    qv_flat = qv_bhtk.reshape(B * H * T, RW)
    pos_flat = pos.reshape(B * H * S)

    sqv_flat = _stage_c_sc(qv_flat, pos_flat)

    xl_bh = _stage_d(sqv_flat.reshape(B * H, S, RW))

    ol_flat = _stage_e_sc(xl_bh.reshape(B * H * S, RW), pos_flat)

    ol4 = ol_flat.reshape(B * H, N_HASHES, T, RW)
    w_out = out_kernel.reshape(H, K, D)
    b_out = out_bias.reshape(1, D)

    return _stage_f(ol4, w_out, b_out, qlen)


# speedup vs baseline: 8.5902x; 1.1050x over previous
import functools

import jax
import jax.numpy as jnp
from jax import lax
from jax.experimental import pallas as pl
from jax.experimental.pallas import tpu as pltpu
from jax.experimental.pallas import tpu_sc as plsc

HIDDEN_DIM = 1024
NUM_HEADS = 16
HEAD_DIM = 64
CHUNK_LEN = 128
N_HASHES = 2
N_BUCKETS = 64

H = NUM_HEADS
K = HEAD_DIM
D = HIDDEN_DIM
T = 4224
S = N_HASHES * T
NCH = S // CHUNK_LEN
TCH = T // CHUNK_LEN
RW = 2 * K


def _proj_body(xq_ref, xv_ref, wq_ref, bq_ref, wv_ref, bv_ref, qv_out):
    x = xq_ref[0]
    q = jnp.dot(x, wq_ref[...], preferred_element_type=jnp.float32) + bq_ref[...]
    xv = xv_ref[0]
    v = jnp.dot(xv, wv_ref[...], preferred_element_type=jnp.float32) + bv_ref[...]
    for h in range(H):
        qv_out[0, h, :, 0:K] = q[:, h * K:(h + 1) * K]
        qv_out[0, h, :, K:RW] = v[:, h * K:(h + 1) * K]


def _stage_a(xq, xv, wq, bq, wv, bv, interpret=False):
    B = xq.shape[0]
    TB = 528
    grid = (B, T // TB)
    return pl.pallas_call(
        _proj_body,
        grid=grid,
        in_specs=[
            pl.BlockSpec((1, TB, D), lambda b, s: (b, s, 0)),
            pl.BlockSpec((1, TB, D), lambda b, s: (b, s, 0)),
            pl.BlockSpec((D, D), lambda b, s: (0, 0)),
            pl.BlockSpec((1, D), lambda b, s: (0, 0)),
            pl.BlockSpec((D, D), lambda b, s: (0, 0)),
            pl.BlockSpec((1, D), lambda b, s: (0, 0)),
        ],
        out_specs=pl.BlockSpec((1, H, TB, RW), lambda b, s: (b, 0, s, 0)),
        out_shape=jax.ShapeDtypeStruct((B, H, T, RW), jnp.float32),
        interpret=interpret,
    )(xq, xv, wq, bq, wv, bv)


def _sort_body(qv_ref, r_ref, pos_ref):
    f32 = jnp.float32
    q = qv_ref[0, 0][:, 0:K]
    rot = jnp.dot(q, r_ref[...], preferred_element_type=f32)

    r64 = lax.broadcasted_iota(jnp.int32, (64, 64), 0)
    c64 = lax.broadcasted_iota(jnp.int32, (64, 64), 1)
    u_incl = (r64 <= c64).astype(f32)
    u_strict = (r64 < c64).astype(f32)
    r128 = lax.broadcasted_iota(jnp.int32, (128, 128), 0)
    c128 = lax.broadcasted_iota(jnp.int32, (128, 128), 1)
    l_strict = (c128 < r128).astype(f32)

    for j in range(N_HASHES):
        rj = rot[:, 32 * j:32 * (j + 1)]
        vals = jnp.concatenate([rj, -rj], axis=1)
        m = jnp.max(vals, axis=1, keepdims=True)
        oh = (vals == m).astype(f32)
        cum = jnp.dot(oh, u_incl, preferred_element_type=f32)
        first = oh * (cum == 1.0).astype(f32)

        running = jnp.zeros((1, 64), f32)
        offs = []
        for c in range(TCH):
            ohc = first[c * 128:(c + 1) * 128, :]
            offs.append(running)
            running = running + jnp.sum(ohc, axis=0, keepdims=True)
        starts = jnp.dot(running, u_strict, preferred_element_type=f32)
        starts = starts + jnp.float32(j * T)
        for c in range(TCH):
            ohc = first[c * 128:(c + 1) * 128, :]
            pe = jnp.dot(l_strict, ohc, preferred_element_type=f32)
            pe = pe + offs[c] + starts
            posc = jnp.sum(ohc * pe, axis=1, keepdims=True)
            pos_ref[0, 0, pl.ds(j * T + c * 128, 128), :] = posc.astype(
                jnp.int32)


def _stage_b(qv_bhtk, rots, interpret=False):
    B = qv_bhtk.shape[0]
    grid = (B * H,)
    return pl.pallas_call(
        _sort_body,
        grid=grid,
        in_specs=[
            pl.BlockSpec((1, 1, T, RW), lambda w: (w // H, w % H, 0, 0)),
            pl.BlockSpec((K, 64), lambda w: (0, 0)),
        ],
        out_specs=pl.BlockSpec((1, 1, S, 1), lambda w: (w // H, w % H, 0, 0)),
        out_shape=jax.ShapeDtypeStruct((B, H, S, 1), jnp.int32),
        interpret=interpret,
    )(qv_bhtk, rots)


_GR = 3
_GRR = _GR * 128
_NG = TCH // _GR


def _stage_c_sc(qv3, pos3):
    BH = qv3.shape[0]
    mesh = plsc.VectorSubcoreMesh(core_axis_name="c", subcore_axis_name="s")

    @functools.partial(
        pl.kernel,
        mesh=mesh,
        out_type=jax.ShapeDtypeStruct((BH, S, RW), jnp.float32),
        scratch_types=[
            pltpu.VMEM((NCH, 128), jnp.int32),
            pltpu.VMEM((_GRR, RW), jnp.float32),
            pltpu.VMEM((_GRR, RW), jnp.float32),
            pltpu.SemaphoreType.DMA,
            pltpu.SemaphoreType.DMA,
        ],
    )
    def sc_fwd(qv_hbm, pos_hbm, sqv_hbm, idx_vm, rows0, rows1, sem0, sem1):
        wid = lax.axis_index("s") * 2 + lax.axis_index("c")
        pltpu.sync_copy(pos_hbm.at[wid], idx_vm)
        pending = {0: [], 1: []}
        bufs = (rows0, rows1)
        sems = (sem0, sem1)
        for hh in range(N_HASHES):
            for g in range(_NG):
                par = g % 2
                rb, sem = bufs[par], sems[par]
                for cp in pending[par]:
                    cp.wait()
                pending[par] = []
                pltpu.sync_copy(qv_hbm.at[wid, pl.ds(g * _GRR, _GRR)], rb)
                for k in range(_GR):
                    c = hh * TCH + g * _GR + k
                    cp = pltpu.make_async_copy(
                        rb.at[pl.ds(k * 128, 128)],
                        sqv_hbm.at[wid].at[idx_vm.at[c]],
                        sem)
                    cp.start()
                    pending[par].append(cp)
        for par in (0, 1):
            for cp in pending[par]:
                cp.wait()

    return sc_fwd(qv3, pos3)


def _attn_body(sqv_ref, xl_ref):
    for c in range(NCH):
        p = (c + NCH - 1) % NCH
        cur = sqv_ref[0, pl.ds(c * 128, 128), :]
        prv = sqv_ref[0, pl.ds(p * 128, 128), :]
        bq = cur[:, 0:K]
        d1 = lax.dot_general(bq, cur[:, 0:K], (((1,), (1,)), ((), ())),
                             preferred_element_type=jnp.float32) * 8.0
        d2 = lax.dot_general(bq, prv[:, 0:K], (((1,), (1,)), ((), ())),
                             preferred_element_type=jnp.float32) * 8.0
        m = jnp.maximum(jnp.max(d1, axis=1, keepdims=True),
                        jnp.max(d2, axis=1, keepdims=True))
        e1 = jnp.exp(d1 - m)
        e2 = jnp.exp(d2 - m)
        ssum = (jnp.sum(e1, axis=1, keepdims=True)
                + jnp.sum(e2, axis=1, keepdims=True))
        lse = m + jnp.log(ssum)
        inv = 1.0 / ssum
        xo = (jnp.dot(e1, cur[:, K:RW], preferred_element_type=jnp.float32)
              + jnp.dot(e2, prv[:, K:RW], preferred_element_type=jnp.float32)
              ) * inv
        xl_ref[0, pl.ds(c * 128, 128), 0:K] = xo
        xl_ref[0, pl.ds(c * 128, 128), K:RW] = jnp.broadcast_to(lse, (128, K))


def _stage_d(sqv, interpret=False):
    BH = sqv.shape[0]
    return pl.pallas_call(
        _attn_body,
        grid=(BH,),
        in_specs=[pl.BlockSpec((1, S, RW), lambda w: (w, 0, 0))],
        out_specs=pl.BlockSpec((1, S, RW), lambda w: (w, 0, 0)),
        out_shape=jax.ShapeDtypeStruct((BH, S, RW), jnp.float32),
        interpret=interpret,
    )(sqv)


def _stage_e_sc(xl3, pos3):
    BH = xl3.shape[0]
    NGRP = S // _GRR
    mesh = plsc.VectorSubcoreMesh(core_axis_name="c", subcore_axis_name="s")

    @functools.partial(
        pl.kernel,
        mesh=mesh,
        out_type=jax.ShapeDtypeStruct((BH, S, RW), jnp.float32),
        scratch_types=[
            pltpu.VMEM((NCH, 128), jnp.int32),
            pltpu.VMEM((_GRR, RW), jnp.float32),
            pltpu.VMEM((_GRR, RW), jnp.float32),
            pltpu.SemaphoreType.DMA,
            pltpu.SemaphoreType.DMA,
        ],
    )
    def sc_bwd(xl_hbm, pos_hbm, ol_hbm, idx_vm, rows0, rows1, sem0, sem1):
        wid = lax.axis_index("s") * 2 + lax.axis_index("c")
        pltpu.sync_copy(pos_hbm.at[wid], idx_vm)
        bufs = (rows0, rows1)
        sems = (sem0, sem1)
        pending = {0: [], 1: []}
        for g in range(NGRP):
            par = g % 2
            rb, sem = bufs[par], sems[par]
            for k in range(_GR):
                c = g * _GR + k
                cp = pltpu.make_async_copy(
                    xl_hbm.at[wid].at[idx_vm.at[c]],
                    rb.at[pl.ds(k * 128, 128)],
                    sem)
                cp.start()
                pending[par].append(cp)
            if g > 0:
                prv = 1 - par
                for cp in pending[prv]:
                    cp.wait()
                pending[prv] = []
                pltpu.sync_copy(bufs[prv],
                                ol_hbm.at[wid, pl.ds((g - 1) * _GRR, _GRR)])
        for cp in pending[(NGRP - 1) % 2]:
            cp.wait()
        pltpu.sync_copy(bufs[(NGRP - 1) % 2],
                        ol_hbm.at[wid, pl.ds((NGRP - 1) * _GRR, _GRR)])

    return sc_bwd(xl3, pos3)


def _out_body(ol_ref, w_ref, b_ref, out_ref):
    h = pl.program_id(2)
    o0 = ol_ref[0, 0, :, 0:K]
    o1 = ol_ref[0, 1, :, 0:K]
    l0 = ol_ref[0, 0, :, K:K + 1]
    l1 = ol_ref[0, 1, :, K:K + 1]
    m = jnp.maximum(l0, l1)
    e0 = jnp.exp(l0 - m)
    e1 = jnp.exp(l1 - m)
    comb = (o0 * e0 + o1 * e1) / (e0 + e1)
    part = jnp.dot(comb, w_ref[0], preferred_element_type=jnp.float32)

    @pl.when(h == 0)
    def _():
        out_ref[0] = part + b_ref[...]

    @pl.when(h != 0)
    def _():
        out_ref[0] = out_ref[0] + part


def _stage_f(ol4, w_out, b_out, qlen, interpret=False):
    BH = ol4.shape[0]
    B = BH // H
    TBF = 512
    grid = (B, qlen // TBF, H)
    return pl.pallas_call(
        _out_body,
        grid=grid,
        in_specs=[
            pl.BlockSpec((1, 2, TBF, RW), lambda b, s, h: (b * H + h, 0, s, 0)),
            pl.BlockSpec((1, K, D), lambda b, s, h: (h, 0, 0)),
            pl.BlockSpec((1, D), lambda b, s, h: (0, 0)),
        ],
        out_specs=pl.BlockSpec((1, TBF, D), lambda b, s, h: (b, s, 0)),
        out_shape=jax.ShapeDtypeStruct((B, qlen, D), jnp.float32),
        interpret=interpret,
    )(ol4, w_out, b_out)


def kernel(inputs_q, inputs_k, inputs_v, q_kernel, q_bias, k_kernel, k_bias,
           v_kernel, v_bias, out_kernel, out_bias, step):
    del step, inputs_k, k_kernel, k_bias
    B, qlen, _ = inputs_q.shape
    extra = CHUNK_LEN - qlen % CHUNK_LEN
    pad = ((0, 0), (0, extra), (0, 0))
    xq = jnp.pad(inputs_q, pad)
    xv = jnp.pad(inputs_v, pad)

    rots = jax.random.normal(
        jax.random.PRNGKey(0),
        (K, N_HASHES, N_BUCKETS // 2)).astype(jnp.float32).reshape(K, 64)

    wq = q_kernel.reshape(D, H * K)
    wv = v_kernel.reshape(D, H * K)
    bq = q_bias.reshape(1, H * K)
    bv = v_bias.reshape(1, H * K)

    qv_bhtk = _stage_a(xq, xv, wq, bq, wv, bv)
    pos = _stage_b(qv_bhtk, rots)

    qv3 = qv_bhtk.reshape(B * H, T, RW)
    pos3 = pos.reshape(B * H, NCH, 128)

    sqv = _stage_c_sc(qv3, pos3)

    xl_bh = _stage_d(sqv)

    ol = _stage_e_sc(xl_bh, pos3)

    ol4 = ol.reshape(B * H, N_HASHES, T, RW)
    w_out = out_kernel.reshape(H, K, D)
    b_out = out_bias.reshape(1, D)

    return _stage_f(ol4, w_out, b_out, qlen)
